# baseline (device time: 15076 ns/iter reference)
import jax
import jax.numpy as jnp
from jax import lax
from jax.experimental import pallas as pl
from jax.experimental.pallas import tpu as pltpu

N_DEV = 4


def kernel(q, k, v):
    s_per, d = q.shape
    scale = 1.0 / (d ** 0.5)

    def body(q_ref, k_ref, v_ref, out_ref,
             kloc, vloc, kbuf, vbuf, ksend, krecv, vsend, vrecv):
        my = lax.axis_index("i")

        kloc[:, :] = k_ref[:, :].astype(jnp.bfloat16)
        vloc[:, :] = v_ref[:, :].astype(jnp.bfloat16)
        q_bf = (q_ref[:, :] * scale).astype(jnp.bfloat16)

        barrier_sem = pltpu.get_barrier_semaphore()
        for off in range(1, N_DEV):
            pl.semaphore_signal(
                barrier_sem, inc=1,
                device_id=((my + off) % N_DEV,),
                device_id_type=pl.DeviceIdType.MESH,
            )
        pl.semaphore_wait(barrier_sem, N_DEV - 1)

        k_rdmas, v_rdmas = {}, {}
        for off in (1, 3, 2):
            tgt = (my + off) % N_DEV
            k_rdmas[off] = pltpu.make_async_remote_copy(
                src_ref=kloc, dst_ref=kbuf.at[off - 1],
                send_sem=ksend.at[off - 1], recv_sem=krecv.at[off - 1],
                device_id=(tgt,), device_id_type=pl.DeviceIdType.MESH,
            )
            k_rdmas[off].start()
        for off in (1, 3, 2):
            tgt = (my + off) % N_DEV
            v_rdmas[off] = pltpu.make_async_remote_copy(
                src_ref=vloc, dst_ref=vbuf.at[off - 1],
                send_sem=vsend.at[off - 1], recv_sem=vrecv.at[off - 1],
                device_id=(tgt,), device_id_type=pl.DeviceIdType.MESH,
            )
            v_rdmas[off].start()

        m = jnp.full((s_per, 1), -jnp.inf, dtype=jnp.float32)
        l = jnp.zeros((s_per, 1), dtype=jnp.float32)
        acc = jnp.zeros((s_per, d), dtype=jnp.float32)

        def scores_update(m, l, k_c):
            s = lax.dot_general(
                q_bf, k_c, (((1,), (1,)), ((), ())),
                preferred_element_type=jnp.float32,
            )
            m_new = jnp.maximum(m, jnp.max(s, axis=1, keepdims=True))
            alpha = jnp.exp(m - m_new)
            p = jnp.exp(s - m_new)
            l = l * alpha + jnp.sum(p, axis=1, keepdims=True)
            return m_new, l, alpha, p.astype(jnp.bfloat16)

        def acc_update(acc, alpha, p, v_c):
            return acc * alpha + lax.dot(
                p, v_c, preferred_element_type=jnp.float32
            )

        m, l, alpha, p = scores_update(m, l, kloc[...])
        acc = acc_update(acc, alpha, p, vloc[...])

        for off in (1, 3, 2):
            slot = off - 1
            k_rdmas[off].wait_recv()
            m, l, alpha, p = scores_update(m, l, kbuf[slot])
            v_rdmas[off].wait_recv()
            acc = acc_update(acc, alpha, p, vbuf[slot])

        out_ref[:, :] = acc / l

        for off in (1, 3, 2):
            k_rdmas[off].wait_send()
            v_rdmas[off].wait_send()

    return pl.pallas_call(
        body,
        out_shape=jax.ShapeDtypeStruct((s_per, d), jnp.float32),
        in_specs=[
            pl.BlockSpec(memory_space=pltpu.VMEM),
            pl.BlockSpec(memory_space=pltpu.VMEM),
            pl.BlockSpec(memory_space=pltpu.VMEM),
        ],
        out_specs=pl.BlockSpec(memory_space=pltpu.VMEM),
        scratch_shapes=[
            pltpu.VMEM((s_per, d), jnp.bfloat16),
            pltpu.VMEM((s_per, d), jnp.bfloat16),
            pltpu.VMEM((N_DEV - 1, s_per, d), jnp.bfloat16),
            pltpu.VMEM((N_DEV - 1, s_per, d), jnp.bfloat16),
            pltpu.SemaphoreType.DMA((N_DEV - 1,)),
            pltpu.SemaphoreType.DMA((N_DEV - 1,)),
            pltpu.SemaphoreType.DMA((N_DEV - 1,)),
            pltpu.SemaphoreType.DMA((N_DEV - 1,)),
        ],
        compiler_params=pltpu.CompilerParams(collective_id=0),
    )(q, k, v)
